# baseline (speedup 1.0000x reference)
import functools

import jax
import jax.numpy as jnp
from jax import lax
from jax.experimental import pallas as pl
from jax.experimental.pallas import tpu as pltpu
from jax.experimental.pallas import tpu_sc as plsc

DIM = 128
M = 200
B = 4096
NC, NS = 2, 16
NW = NC * NS
EPW = B // NW
MA = 128
MB = M - MA


def _sc_embed(emb16, emb32, uidx, items):
    mesh = plsc.VectorSubcoreMesh(core_axis_name="c", subcore_axis_name="s")
    out_type = (
        jax.ShapeDtypeStruct((B, DIM), jnp.float32),
        jax.ShapeDtypeStruct((B, DIM), jnp.float32),
    )

    @functools.partial(
        pl.kernel,
        mesh=mesh,
        out_type=out_type,
        compiler_params=pltpu.CompilerParams(
            needs_layout_passes=False, use_tc_tiling_on_sc=False),
        scratch_types=[
            pltpu.VMEM((EPW * M,), jnp.int32),
            pltpu.VMEM((MA, DIM // 2), jnp.int32),
            pltpu.VMEM((MB, DIM // 2), jnp.int32),
            pltpu.VMEM((MA, DIM // 2), jnp.int32),
            pltpu.VMEM((MB, DIM // 2), jnp.int32),
            pltpu.VMEM((EPW, DIM), jnp.float32),
            pltpu.VMEM((EPW,), jnp.int32),
            pltpu.VMEM((EPW, DIM), jnp.float32),
            pltpu.SemaphoreType.DMA,
            pltpu.SemaphoreType.DMA,
            pltpu.SemaphoreType.DMA,
        ],
    )
    def k(emb16_hbm, emb32_hbm, uidx_hbm, items_hbm, eu_hbm, ev_hbm,
          idx_v, rows_a0, rows_b0, rows_a1, rows_b1, eu_v, it_v, ev_v,
          sem0, sem1, semx):
        wid = lax.axis_index("s") * NC + lax.axis_index("c")
        base = wid * EPW

        pltpu.sync_copy(items_hbm.at[pl.ds(base, EPW)], it_v)
        pltpu.async_copy(emb32_hbm.at[it_v], ev_v, semx).wait()
        pltpu.sync_copy(ev_v, ev_hbm.at[pl.ds(base, EPW)])

        pltpu.sync_copy(uidx_hbm.at[pl.ds(base * M, EPW * M)], idx_v)

        def issue(e, ra, rb, sem):
            off_a = pl.multiple_of(e * M, 8)
            off_b = pl.multiple_of(e * M + MA, 8)
            pltpu.async_copy(emb16_hbm.at[idx_v.at[pl.ds(off_a, MA)]], ra, sem)
            pltpu.async_copy(emb16_hbm.at[idx_v.at[pl.ds(off_b, MB)]], rb, sem)

        def wait_set(ra, rb, sem):
            pltpu.make_async_copy(
                emb16_hbm.at[idx_v.at[pl.ds(0, MA)]], ra, sem).wait()
            pltpu.make_async_copy(
                emb16_hbm.at[idx_v.at[pl.ds(0, MB)]], rb, sem).wait()

        lane = lax.iota(jnp.int32, 16)

        def accum(e, ra, rb):
            def add_row(rows, r, a):
                out = []
                for g in range(4):
                    packed = plsc.bitcast(rows[r, pl.ds(g * 16, 16)],
                                          jnp.bfloat16)
                    lo, hi = plsc.unpack(packed,
                                         format=plsc.PackFormat.INTERLEAVED)
                    out.append(a[2 * g] + lo)
                    out.append(a[2 * g + 1] + hi)
                return tuple(out)

            def body_a(r2, accs):
                return add_row(ra, r2 * 2 + 1, add_row(ra, r2 * 2, accs))

            accs = lax.fori_loop(
                0, MA // 2, body_a,
                tuple(jnp.zeros((16,), jnp.float32) for _ in range(8)))

            def body_b(r2, accs):
                return add_row(rb, r2 * 2 + 1, add_row(rb, r2 * 2, accs))

            accs = lax.fori_loop(0, MB // 2, body_b, accs)
            erow = jnp.full((16,), e, jnp.int32)
            for g in range(4):
                plsc.store_scatter(eu_v, [erow, g * 32 + 2 * lane],
                                   accs[2 * g] * (1.0 / M))
                plsc.store_scatter(eu_v, [erow, g * 32 + 2 * lane + 1],
                                   accs[2 * g + 1] * (1.0 / M))

        issue(0, rows_a0, rows_b0, sem0)

        @pl.loop(0, EPW // 2)
        def per_pair(g):
            e0 = g * 2
            issue(e0 + 1, rows_a1, rows_b1, sem1)
            wait_set(rows_a0, rows_b0, sem0)
            accum(e0, rows_a0, rows_b0)

            @pl.when(e0 + 2 < EPW)
            def _():
                issue(e0 + 2, rows_a0, rows_b0, sem0)

            wait_set(rows_a1, rows_b1, sem1)
            accum(e0 + 1, rows_a1, rows_b1)

        pltpu.sync_copy(eu_v, eu_hbm.at[pl.ds(base, EPW)])

    return k(emb16, emb32, uidx, items)


def _tc_score_body(eu_ref, ev_ref, y_ref, s_ref, loss_ref):
    d = jnp.sum(eu_ref[...] * ev_ref[...], axis=1, keepdims=True)
    s = jax.nn.sigmoid(d)
    s_ref[...] = s
    y = y_ref[...]
    eps = 1e-12
    sc = jnp.clip(s, eps, 1.0 - eps)
    bl = y * jnp.log(sc) + (1.0 - y) * jnp.log(1.0 - sc)
    loss_ref[...] = -jnp.sum(bl, axis=(0, 1), keepdims=True) * (1.0 / B)


def _tc_score(eu, ev, y):
    return pl.pallas_call(
        _tc_score_body,
        out_shape=(
            jax.ShapeDtypeStruct((B, 1), jnp.float32),
            jax.ShapeDtypeStruct((1, 1), jnp.float32),
        ),
    )(eu, ev, y)


def kernel(entity_emb, items, labels, user_triple_set, item_triple_set):
    uidx = user_triple_set[0, 0].astype(jnp.int32).reshape(-1)
    it = items.astype(jnp.int32)
    emb16 = entity_emb.astype(jnp.bfloat16)
    emb16p = lax.bitcast_convert_type(
        emb16.reshape(entity_emb.shape[0], DIM // 2, 2), jnp.int32)
    eu, ev = _sc_embed(emb16p, entity_emb, uidx, it)
    y = labels.astype(jnp.float32).reshape(B, 1)
    s, loss = _tc_score(eu, ev, y)
    return s.reshape(B), loss[0, 0]


# speedup vs baseline: 2.6236x; 2.6236x over previous
import functools

import jax
import jax.numpy as jnp
from jax import lax
from jax.experimental import pallas as pl
from jax.experimental.pallas import tpu as pltpu
from jax.experimental.pallas import tpu_sc as plsc

DIM = 128
M = 200
B = 4096
NC, NS = 2, 16
NW = NC * NS
EPW = B // NW
MA = 128
MB = M - MA


def _sc_embed(emb16, emb32, uidx, items):
    mesh = plsc.VectorSubcoreMesh(core_axis_name="c", subcore_axis_name="s")
    out_type = (
        jax.ShapeDtypeStruct((B, DIM), jnp.float32),
        jax.ShapeDtypeStruct((B, DIM), jnp.float32),
    )

    @functools.partial(
        pl.kernel,
        mesh=mesh,
        out_type=out_type,
        compiler_params=pltpu.CompilerParams(
            needs_layout_passes=False, use_tc_tiling_on_sc=False),
        scratch_types=[
            pltpu.VMEM((EPW * M,), jnp.int32),
            pltpu.VMEM((MA, DIM // 2), jnp.int32),
            pltpu.VMEM((MB, DIM // 2), jnp.int32),
            pltpu.VMEM((MA, DIM // 2), jnp.int32),
            pltpu.VMEM((MB, DIM // 2), jnp.int32),
            pltpu.VMEM((EPW, DIM), jnp.float32),
            pltpu.VMEM((EPW,), jnp.int32),
            pltpu.VMEM((EPW, DIM), jnp.float32),
            pltpu.SemaphoreType.DMA,
            pltpu.SemaphoreType.DMA,
            pltpu.SemaphoreType.DMA,
        ],
    )
    def k(emb16_hbm, emb32_hbm, uidx_hbm, items_hbm, eu_hbm, ev_hbm,
          idx_v, rows_a0, rows_b0, rows_a1, rows_b1, eu_v, it_v, ev_v,
          sem0, sem1, semx):
        wid = lax.axis_index("s") * NC + lax.axis_index("c")
        base = wid * EPW

        pltpu.sync_copy(items_hbm.at[pl.ds(base, EPW)], it_v)
        pltpu.async_copy(emb32_hbm.at[it_v], ev_v, semx).wait()
        pltpu.sync_copy(ev_v, ev_hbm.at[pl.ds(base, EPW)])

        pltpu.sync_copy(uidx_hbm.at[pl.ds(base * M, EPW * M)], idx_v)

        def issue(e, ra, rb, sem):
            off_a = pl.multiple_of(e * M, 8)
            off_b = pl.multiple_of(e * M + MA, 8)
            pltpu.async_copy(emb16_hbm.at[idx_v.at[pl.ds(off_a, MA)]], ra, sem)
            pltpu.async_copy(emb16_hbm.at[idx_v.at[pl.ds(off_b, MB)]], rb, sem)

        def wait_set(ra, rb, sem):
            pltpu.make_async_copy(
                emb16_hbm.at[idx_v.at[pl.ds(0, MA)]], ra, sem).wait()
            pltpu.make_async_copy(
                emb16_hbm.at[idx_v.at[pl.ds(0, MB)]], rb, sem).wait()

        def accum(e, ra, rb):
            def add_row(rows, r, a):
                out = list(a)
                for g in range(4):
                    packed = plsc.bitcast(rows[r, pl.ds(g * 16, 16)],
                                          jnp.bfloat16)
                    lo, hi = plsc.unpack(packed,
                                         format=plsc.PackFormat.INTERLEAVED)
                    out[g] = out[g] + lo
                    out[g + 4] = out[g + 4] + hi
                return tuple(out)

            def body_a(r2, accs):
                return add_row(ra, r2 * 2 + 1, add_row(ra, r2 * 2, accs))

            accs = lax.fori_loop(
                0, MA // 2, body_a,
                tuple(jnp.zeros((16,), jnp.float32) for _ in range(8)))

            def body_b(r2, accs):
                return add_row(rb, r2 * 2 + 1, add_row(rb, r2 * 2, accs))

            accs = lax.fori_loop(0, MB // 2, body_b, accs)
            for g in range(4):
                eu_v[e, pl.ds(g * 16, 16)] = accs[g] * (1.0 / M)
                eu_v[e, pl.ds(64 + g * 16, 16)] = accs[g + 4] * (1.0 / M)

        issue(0, rows_a0, rows_b0, sem0)

        @pl.loop(0, EPW // 2)
        def per_pair(g):
            e0 = g * 2
            issue(e0 + 1, rows_a1, rows_b1, sem1)
            wait_set(rows_a0, rows_b0, sem0)
            accum(e0, rows_a0, rows_b0)

            @pl.when(e0 + 2 < EPW)
            def _():
                issue(e0 + 2, rows_a0, rows_b0, sem0)

            wait_set(rows_a1, rows_b1, sem1)
            accum(e0 + 1, rows_a1, rows_b1)

        pltpu.sync_copy(eu_v, eu_hbm.at[pl.ds(base, EPW)])

    return k(emb16, emb32, uidx, items)


PACK_ROWS = 4096


def _tc_pack_body(x_ref, o_ref):
    x = x_ref[...]
    a = x[:, :64].astype(jnp.bfloat16)
    b = x[:, 64:].astype(jnp.bfloat16)
    au = lax.bitcast_convert_type(a, jnp.uint16).astype(jnp.uint32)
    bu = lax.bitcast_convert_type(b, jnp.uint16).astype(jnp.uint32)
    o_ref[...] = (au | (bu << 16)).astype(jnp.int32)


def _tc_pack(x):
    n = x.shape[0]
    grid = (n + PACK_ROWS - 1) // PACK_ROWS
    return pl.pallas_call(
        _tc_pack_body,
        grid=(grid,),
        in_specs=[pl.BlockSpec((PACK_ROWS, DIM), lambda i: (i, 0))],
        out_specs=pl.BlockSpec((PACK_ROWS, DIM // 2), lambda i: (i, 0)),
        out_shape=jax.ShapeDtypeStruct((n, DIM // 2), jnp.int32),
    )(x)


def _tc_score_body(eu_ref, ev_ref, y_ref, s_ref, loss_ref):
    d = jnp.sum(eu_ref[...] * ev_ref[...], axis=1, keepdims=True)
    s = jax.nn.sigmoid(d)
    s_ref[...] = s
    y = y_ref[...]
    eps = 1e-12
    sc = jnp.clip(s, eps, 1.0 - eps)
    bl = y * jnp.log(sc) + (1.0 - y) * jnp.log(1.0 - sc)
    loss_ref[...] = -jnp.sum(bl, axis=(0, 1), keepdims=True) * (1.0 / B)


def _tc_score(eu, ev, y):
    return pl.pallas_call(
        _tc_score_body,
        out_shape=(
            jax.ShapeDtypeStruct((B, 1), jnp.float32),
            jax.ShapeDtypeStruct((1, 1), jnp.float32),
        ),
    )(eu, ev, y)


def kernel(entity_emb, items, labels, user_triple_set, item_triple_set):
    uidx = user_triple_set[0, 0].astype(jnp.int32).reshape(-1)
    it = items.astype(jnp.int32)
    emb16p = _tc_pack(entity_emb)
    eu, ev = _sc_embed(emb16p, entity_emb, uidx, it)
    y = labels.astype(jnp.float32).reshape(B, 1)
    s, loss = _tc_score(eu, ev, y)
    return s.reshape(B), loss[0, 0]


# speedup vs baseline: 2.8167x; 1.0736x over previous
import functools

import jax
import jax.numpy as jnp
from jax import lax
from jax.experimental import pallas as pl
from jax.experimental.pallas import tpu as pltpu
from jax.experimental.pallas import tpu_sc as plsc

DIM = 128
M = 200
B = 4096
NC, NS = 2, 16
NW = NC * NS
EPW = B // NW
MA = 128
MB = M - MA


def _sc_embed(emb, uidx, items):
    mesh = plsc.VectorSubcoreMesh(core_axis_name="c", subcore_axis_name="s")
    out_type = (
        jax.ShapeDtypeStruct((B, DIM), jnp.float32),
        jax.ShapeDtypeStruct((B, DIM), jnp.float32),
    )

    @functools.partial(
        pl.kernel,
        mesh=mesh,
        out_type=out_type,
        scratch_types=[
            pltpu.VMEM((EPW * M,), jnp.int32),
            pltpu.VMEM((MA, DIM), jnp.float32),
            pltpu.VMEM((MB, DIM), jnp.float32),
            pltpu.VMEM((MA, DIM), jnp.float32),
            pltpu.VMEM((MB, DIM), jnp.float32),
            pltpu.VMEM((EPW, DIM), jnp.float32),
            pltpu.VMEM((EPW,), jnp.int32),
            pltpu.VMEM((EPW, DIM), jnp.float32),
            pltpu.SemaphoreType.DMA,
            pltpu.SemaphoreType.DMA,
            pltpu.SemaphoreType.DMA,
        ],
    )
    def k(emb_hbm, uidx_hbm, items_hbm, eu_hbm, ev_hbm,
          idx_v, rows_a0, rows_b0, rows_a1, rows_b1, eu_v, it_v, ev_v,
          sem0, sem1, semx):
        wid = lax.axis_index("s") * NC + lax.axis_index("c")
        base = wid * EPW

        pltpu.sync_copy(items_hbm.at[pl.ds(base, EPW)], it_v)
        pltpu.async_copy(emb_hbm.at[it_v], ev_v, semx).wait()
        pltpu.sync_copy(ev_v, ev_hbm.at[pl.ds(base, EPW)])

        pltpu.sync_copy(uidx_hbm.at[pl.ds(base * M, EPW * M)], idx_v)

        def issue(e, ra, rb, sem):
            off_a = pl.multiple_of(e * M, 8)
            off_b = pl.multiple_of(e * M + MA, 8)
            pltpu.async_copy(emb_hbm.at[idx_v.at[pl.ds(off_a, MA)]], ra, sem)
            pltpu.async_copy(emb_hbm.at[idx_v.at[pl.ds(off_b, MB)]], rb, sem)

        def wait_set(ra, rb, sem):
            pltpu.make_async_copy(
                emb_hbm.at[idx_v.at[pl.ds(0, MA)]], ra, sem).wait()
            pltpu.make_async_copy(
                emb_hbm.at[idx_v.at[pl.ds(0, MB)]], rb, sem).wait()

        def accum(e, ra, rb):
            def body_a(r4, accs):
                a = accs
                for u in range(4):
                    a = tuple(a[c] + ra[r4 * 4 + u, pl.ds(c * 16, 16)]
                              for c in range(8))
                return a

            accs = lax.fori_loop(
                0, MA // 4, body_a,
                tuple(jnp.zeros((16,), jnp.float32) for _ in range(8)))

            def body_b(r4, accs):
                a = accs
                for u in range(4):
                    a = tuple(a[c] + rb[r4 * 4 + u, pl.ds(c * 16, 16)]
                              for c in range(8))
                return a

            accs = lax.fori_loop(0, MB // 4, body_b, accs)
            for c in range(8):
                eu_v[e, pl.ds(c * 16, 16)] = accs[c] * (1.0 / M)

        issue(0, rows_a0, rows_b0, sem0)

        @pl.loop(0, EPW // 2)
        def per_pair(g):
            e0 = g * 2
            issue(e0 + 1, rows_a1, rows_b1, sem1)
            wait_set(rows_a0, rows_b0, sem0)
            accum(e0, rows_a0, rows_b0)

            @pl.when(e0 + 2 < EPW)
            def _():
                issue(e0 + 2, rows_a0, rows_b0, sem0)

            wait_set(rows_a1, rows_b1, sem1)
            accum(e0 + 1, rows_a1, rows_b1)

        pltpu.sync_copy(eu_v, eu_hbm.at[pl.ds(base, EPW)])

    return k(emb, uidx, items)


def _tc_score_body(eu_ref, ev_ref, y_ref, s_ref, loss_ref):
    d = jnp.sum(eu_ref[...] * ev_ref[...], axis=1, keepdims=True)
    s = jax.nn.sigmoid(d)
    s_ref[...] = s
    y = y_ref[...]
    eps = 1e-12
    sc = jnp.clip(s, eps, 1.0 - eps)
    bl = y * jnp.log(sc) + (1.0 - y) * jnp.log(1.0 - sc)
    loss_ref[...] = -jnp.sum(bl, axis=(0, 1), keepdims=True) * (1.0 / B)


def _tc_score(eu, ev, y):
    return pl.pallas_call(
        _tc_score_body,
        out_shape=(
            jax.ShapeDtypeStruct((B, 1), jnp.float32),
            jax.ShapeDtypeStruct((1, 1), jnp.float32),
        ),
    )(eu, ev, y)


def kernel(entity_emb, items, labels, user_triple_set, item_triple_set):
    uidx = user_triple_set[0, 0].astype(jnp.int32).reshape(-1)
    it = items.astype(jnp.int32)
    eu, ev = _sc_embed(entity_emb, uidx, it)
    y = labels.astype(jnp.float32).reshape(B, 1)
    s, loss = _tc_score(eu, ev, y)
    return s.reshape(B), loss[0, 0]
